# Initial kernel scaffold; baseline (speedup 1.0000x reference)
import jax
import jax.numpy as jnp
from jax.experimental import pallas as pl


def kernel(x, edge_index, W1, att_src1, att_dst1, b1, W2, att_src2, att_dst2, b2):
    raise NotImplementedError("write your pallas kernel here")


# speedup vs baseline: 16.8848x; 16.8848x over previous
import functools

import jax
import jax.numpy as jnp
from jax import lax
from jax.experimental import pallas as pl
from jax.experimental.pallas import tpu as pltpu
from jax.experimental.pallas import tpu_sc as plsc

N_NODES = 10000
D_IN = 128
D_HID = 16
D_OUT = 128
N_EDGES = 320000

N_PAD = 10240
E_TOT = N_EDGES + N_NODES
CH = 128
N_WORKERS = 32
CPW = -(-E_TOT // (N_WORKERS * CH))
E_PAD = N_WORKERS * CPW * CH
ROWS_PT = N_PAD // 16


def _tc_dense_a(x_ref, w_ref, asrc_ref, adst_ref, h_ref, as_ref, ad_ref):
    h = jnp.dot(x_ref[...], w_ref[...], preferred_element_type=jnp.float32)
    h_ref[...] = h
    as_ref[...] = jnp.sum(h * asrc_ref[...], axis=1)
    ad_ref[...] = jnp.sum(h * adst_ref[...], axis=1)


def _tc_mid(s_ref, b1_ref, w2_ref, asrc_ref, adst_ref, h2_ref, as_ref, ad_ref):
    s = s_ref[0] + s_ref[1]
    num = s[:, :D_HID]
    den = s[:, D_HID:D_HID + 1]
    hin = jnp.maximum(num / (den + 1e-16) + b1_ref[...], 0.0)
    h2 = jnp.dot(hin, w2_ref[...], preferred_element_type=jnp.float32)
    h2_ref[...] = h2
    as_ref[...] = jnp.sum(h2 * asrc_ref[...], axis=1)
    ad_ref[...] = jnp.sum(h2 * adst_ref[...], axis=1)


def _tc_final(s_ref, b2_ref, out_ref):
    s = s_ref[0] + s_ref[1]
    num = s[:, :D_OUT]
    den = s[:, D_OUT:D_OUT + 1]
    out_ref[...] = num / (den + 1e-16) + b2_ref[...]


_SC_MESH = plsc.VectorSubcoreMesh(core_axis_name="c", subcore_axis_name="s")
_SC_PARAMS = pltpu.CompilerParams(
    needs_layout_passes=False, use_tc_tiling_on_sc=False)


def _sc_w_body(src_hbm, dst_hbm, as_hbm, ad_hbm, w_out,
               as_v, ad_v, sidx_v, didx_v, wbuf_v):
    c = lax.axis_index("c")
    s = lax.axis_index("s")
    pltpu.sync_copy(as_hbm, as_v)
    pltpu.sync_copy(ad_hbm, ad_v)
    wbase = (c * 16 + s) * (CPW * CH)

    def chunk(g, carry):
        base = wbase + g * CH
        pltpu.sync_copy(src_hbm.at[pl.ds(base, CH)], sidx_v)
        pltpu.sync_copy(dst_hbm.at[pl.ds(base, CH)], didx_v)
        for j in range(CH // 16):
            sv = sidx_v[pl.ds(j * 16, 16)]
            dv = didx_v[pl.ds(j * 16, 16)]
            v = plsc.load_gather(as_v, [sv]) + plsc.load_gather(ad_v, [dv])
            v = jnp.where(v >= 0, v, 0.2 * v)
            wbuf_v[pl.ds(j * 16, 16)] = jnp.exp(v)
        pltpu.sync_copy(wbuf_v, w_out.at[pl.ds(base, CH)])
        return carry

    lax.fori_loop(0, CPW, chunk, 0)


_sc_w = pl.kernel(
    _sc_w_body,
    out_type=jax.ShapeDtypeStruct((E_PAD,), jnp.float32),
    mesh=_SC_MESH,
    compiler_params=_SC_PARAMS,
    scratch_types=[
        pltpu.VMEM((N_PAD,), jnp.float32),
        pltpu.VMEM((N_PAD,), jnp.float32),
        pltpu.VMEM((CH,), jnp.int32),
        pltpu.VMEM((CH,), jnp.int32),
        pltpu.VMEM((CH,), jnp.float32),
    ],
)


def _make_sc_agg(C):
    CW = C + 16

    def body(h_hbm, src_hbm, dst_hbm, w_hbm, s_out,
             sidx_v, didx_v, rows_v, srows_v, wbuf_v, s_sh, sem):
        c = lax.axis_index("c")
        s = lax.axis_index("s")

        zv = jnp.zeros((16,), jnp.float32)

        def zrow(i, carry):
            for j in range(CW // 16):
                srows_v[i, pl.ds(j * 16, 16)] = zv
            return carry

        lax.fori_loop(0, CH, zrow, 0)
        r0 = s * ROWS_PT
        for i in range(ROWS_PT // CH):
            pltpu.sync_copy(srows_v, s_sh.at[pl.ds(r0 + i * CH, CH)])
        plsc.subcore_barrier()

        lane0 = lax.iota(jnp.int32, 16) == 0
        wbase = (c * 16 + s) * (CPW * CH)

        def chunk(g, carry):
            base = wbase + g * CH
            pltpu.sync_copy(src_hbm.at[pl.ds(base, CH)], sidx_v)
            pltpu.sync_copy(dst_hbm.at[pl.ds(base, CH)], didx_v.at[0])
            pltpu.sync_copy(w_hbm.at[pl.ds(base, CH)], wbuf_v)
            pltpu.async_copy(h_hbm.at[sidx_v], rows_v, sem).wait()

            def scale_e(e, carry2):
                ws = plsc.load_gather(
                    wbuf_v, [jnp.full((16,), e, jnp.int32)])
                for j in range(C // 16):
                    srows_v[e, pl.ds(j * 16, 16)] = (
                        rows_v[e, pl.ds(j * 16, 16)] * ws)
                srows_v[e, pl.ds(C, 16)] = jnp.where(lane0, ws, 0.0)
                return carry2

            lax.fori_loop(0, CH, scale_e, 0)
            pltpu.sync_copy(srows_v, s_sh.at[didx_v.at[0]], add=True)
            return carry

        lax.fori_loop(0, CPW, chunk, 0)
        plsc.subcore_barrier()
        pltpu.sync_copy(s_sh.at[pl.ds(r0, ROWS_PT)],
                        s_out.at[pl.ds(c * N_PAD + r0, ROWS_PT)])

    return pl.kernel(
        body,
        out_type=jax.ShapeDtypeStruct((2 * N_PAD, CW), jnp.float32),
        mesh=_SC_MESH,
        compiler_params=_SC_PARAMS,
        scratch_types=[
            pltpu.VMEM((CH,), jnp.int32),
            pltpu.VMEM((1, CH), jnp.int32),
            pltpu.VMEM((CH, C), jnp.float32),
            pltpu.VMEM((CH, CW), jnp.float32),
            pltpu.VMEM((CH,), jnp.float32),
            pltpu.VMEM_SHARED((N_PAD, CW), jnp.float32),
            pltpu.SemaphoreType.DMA,
        ],
    )


_sc_agg_1 = _make_sc_agg(D_HID)
_sc_agg_2 = _make_sc_agg(D_OUT)

_dense_a = pl.pallas_call(
    _tc_dense_a,
    out_shape=(
        jax.ShapeDtypeStruct((N_PAD, D_HID), jnp.float32),
        jax.ShapeDtypeStruct((N_PAD,), jnp.float32),
        jax.ShapeDtypeStruct((N_PAD,), jnp.float32),
    ),
)

_mid = pl.pallas_call(
    _tc_mid,
    out_shape=(
        jax.ShapeDtypeStruct((N_PAD, D_OUT), jnp.float32),
        jax.ShapeDtypeStruct((N_PAD,), jnp.float32),
        jax.ShapeDtypeStruct((N_PAD,), jnp.float32),
    ),
)

_final = pl.pallas_call(
    _tc_final,
    out_shape=jax.ShapeDtypeStruct((N_PAD, D_OUT), jnp.float32),
)


def kernel(x, edge_index, W1, att_src1, att_dst1, b1,
           W2, att_src2, att_dst2, b2):
    x = x.astype(jnp.float32)
    loops = jnp.arange(N_NODES, dtype=jnp.int32)
    padv = jnp.full((E_PAD - E_TOT,), N_NODES, dtype=jnp.int32)
    src = jnp.concatenate([edge_index[0].astype(jnp.int32), loops, padv])
    dst = jnp.concatenate([edge_index[1].astype(jnp.int32), loops, padv])

    x_pad = jnp.zeros((N_PAD, D_IN), jnp.float32).at[:N_NODES].set(x)

    h1, as1, ad1 = _dense_a(x_pad, W1, att_src1.reshape(1, D_HID),
                            att_dst1.reshape(1, D_HID))
    w1e = _sc_w(src, dst, as1, ad1)
    s1 = _sc_agg_1(h1, src, dst, w1e).reshape(2, N_PAD, D_HID + 16)
    h2, as2, ad2 = _mid(s1, b1.reshape(1, D_HID), W2,
                        att_src2.reshape(1, D_OUT), att_dst2.reshape(1, D_OUT))
    w2e = _sc_w(src, dst, as2, ad2)
    s2 = _sc_agg_2(h2, src, dst, w2e).reshape(2, N_PAD, D_OUT + 16)
    out = _final(s2, b2.reshape(1, D_OUT))
    return out[:N_NODES]


# speedup vs baseline: 25.4047x; 1.5046x over previous
import jax
import jax.numpy as jnp
from jax import lax
from jax.experimental import pallas as pl
from jax.experimental.pallas import tpu as pltpu
from jax.experimental.pallas import tpu_sc as plsc

N_NODES = 10000
D_IN = 128
D_HID = 16
D_OUT = 128
N_EDGES = 320000

N_PAD = 10112
E_TOT = N_EDGES + N_NODES
CH = 128
N_WORKERS = 32
CPW = -(-E_TOT // (N_WORKERS * CH))
E_PAD = N_WORKERS * CPW * CH
NCHUNKS = N_WORKERS * CPW
SUP = 9
ROWS_PT = N_PAD // 16

_SC_MESH = plsc.VectorSubcoreMesh(core_axis_name="c", subcore_axis_name="s")
_SC_PARAMS = pltpu.CompilerParams(
    needs_layout_passes=False, use_tc_tiling_on_sc=False)


def _tc_dense_a(x_ref, w_ref, asrc_ref, adst_ref, h_ref, as_ref, ad_ref):
    h = jnp.dot(x_ref[...], w_ref[...], preferred_element_type=jnp.float32)
    h_ref[...] = jnp.concatenate(
        [h, jnp.zeros((N_PAD, 16), jnp.float32)], axis=1)
    as_ref[...] = jnp.sum(h * asrc_ref[...], axis=1)
    ad_ref[...] = jnp.sum(h * adst_ref[...], axis=1)


def _tc_mid(s_ref, b1_ref, w2_ref, asrc_ref, adst_ref, h2_ref, as_ref, ad_ref):
    s = s_ref[0] + s_ref[1]
    num = s[:, :D_HID]
    den = s[:, D_HID:D_HID + 1]
    hin = jnp.maximum(num / (den + 1e-16) + b1_ref[...], 0.0)
    h2 = jnp.dot(hin, w2_ref[...], preferred_element_type=jnp.float32)
    h2_ref[...] = jnp.concatenate(
        [h2, jnp.zeros((N_PAD, 16), jnp.float32)], axis=1)
    as_ref[...] = jnp.sum(h2 * asrc_ref[...], axis=1)
    ad_ref[...] = jnp.sum(h2 * adst_ref[...], axis=1)


def _tc_final(s_ref, b2_ref, out_ref):
    s = s_ref[0] + s_ref[1]
    num = s[:, :D_OUT]
    den = s[:, D_OUT:D_OUT + 1]
    out_ref[...] = num / (den + 1e-16) + b2_ref[...]


def _sc_w_body(src_hbm, dst_hbm, as_hbm, ad_hbm, pk_out,
               as_v, ad_v, sd_v, stage_v):
    c = lax.axis_index("c")
    s = lax.axis_index("s")
    wid = c * 16 + s
    pltpu.sync_copy(as_hbm, as_v)
    pltpu.sync_copy(ad_hbm, ad_v)

    def sup(s9, carry):
        ebase = wid * (CPW * CH) + s9 * (SUP * CH)
        pltpu.sync_copy(src_hbm.at[pl.ds(ebase, SUP * CH)], sd_v.at[0])
        pltpu.sync_copy(dst_hbm.at[pl.ds(ebase, SUP * CH)], sd_v.at[1])

        def chunk(k, carry2):
            for j in range(CH // 16):
                sv = sd_v[0, pl.ds(k * CH + j * 16, 16)]
                dv = sd_v[1, pl.ds(k * CH + j * 16, 16)]
                v = (plsc.load_gather(as_v, [sv])
                     + plsc.load_gather(ad_v, [dv]))
                v = jnp.where(v >= 0, v, 0.2 * v)
                w = jnp.exp(v)
                stage_v[k, 0, pl.ds(j * 16, 16)] = sv
                stage_v[k, 1, pl.ds(j * 16, 16)] = dv
                stage_v[k, 2, pl.ds(j * 16, 16)] = plsc.bitcast(w, jnp.int32)
            return carry2

        lax.fori_loop(0, SUP, chunk, 0)
        pltpu.sync_copy(stage_v, pk_out.at[pl.ds(wid * CPW + s9 * SUP, SUP)])
        return carry

    lax.fori_loop(0, CPW // SUP, sup, 0)


_sc_w = pl.kernel(
    _sc_w_body,
    out_type=jax.ShapeDtypeStruct((NCHUNKS, 3, CH), jnp.int32),
    mesh=_SC_MESH,
    compiler_params=_SC_PARAMS,
    scratch_types=[
        pltpu.VMEM((N_PAD,), jnp.float32),
        pltpu.VMEM((N_PAD,), jnp.float32),
        pltpu.VMEM((2, SUP * CH), jnp.int32),
        pltpu.VMEM((SUP, 3, CH), jnp.int32),
    ],
)


def _make_sc_agg(C):
    CW = C + 16

    def _scale(big_v, wring_v, ebuf_v, bg, g3):
        for j in range(CH // 16):
            wring_v[pl.ds(j * 16, 16)] = plsc.bitcast(
                ebuf_v[g3, 2, pl.ds(j * 16, 16)], jnp.float32)
        lane0 = lax.iota(jnp.int32, 16) == 0

        def grp(j2, carry):
            for l in range(16):
                e = j2 * 16 + l
                ws = plsc.load_gather(
                    wring_v, [jnp.full((16,), e, jnp.int32)])
                for k in range(C // 16):
                    big_v[bg, e, pl.ds(k * 16, 16)] = (
                        big_v[bg, e, pl.ds(k * 16, 16)] * ws)
                big_v[bg, e, pl.ds(C, 16)] = jnp.where(lane0, ws, 0.0)
            return carry

        lax.fori_loop(0, CH // 16, grp, 0)

    def body(hp_hbm, pk_hbm, s_out,
             ebuf_v, big_v, wring_v, s_sh, sem_g, sem_s):
        c = lax.axis_index("c")
        s = lax.axis_index("s")
        wid = c * 16 + s
        cbase = wid * CPW
        r0 = s * ROWS_PT

        pltpu.sync_copy(pk_hbm.at[cbase], ebuf_v.at[0])
        pltpu.async_copy(hp_hbm.at[ebuf_v.at[0, 0]], big_v.at[0], sem_g)

        zv = jnp.zeros((16,), jnp.float32)

        def zrow(i, carry):
            for j in range(CW // 16):
                big_v[1, i, pl.ds(j * 16, 16)] = zv
            return carry

        lax.fori_loop(0, CH, zrow, 0)
        for i in range(4):
            pltpu.sync_copy(big_v.at[1], s_sh.at[pl.ds(r0 + i * CH, CH)])
        pltpu.sync_copy(big_v.at[1, pl.ds(0, ROWS_PT - 4 * CH)],
                        s_sh.at[pl.ds(r0 + 4 * CH, ROWS_PT - 4 * CH)])
        plsc.subcore_barrier()

        pltpu.make_async_copy(
            hp_hbm.at[ebuf_v.at[0, 0]], big_v.at[0], sem_g).wait()
        pltpu.sync_copy(pk_hbm.at[cbase + 1], ebuf_v.at[1])
        pltpu.async_copy(hp_hbm.at[ebuf_v.at[1, 0]], big_v.at[1], sem_g)
        _scale(big_v, wring_v, ebuf_v, 0, 0)
        pltpu.async_copy(big_v.at[0], s_sh.at[ebuf_v.at[0, 1]], sem_s,
                         add=True)

        def iter_g(g, carry):
            bg = lax.rem(g, 2)
            nbg = 1 - bg
            g3 = lax.rem(g, 3)
            gn3 = lax.rem(g + 1, 3)
            gp3 = lax.rem(g + 2, 3)
            pltpu.make_async_copy(
                hp_hbm.at[ebuf_v.at[g3, 0]], big_v.at[bg], sem_g).wait()
            pltpu.make_async_copy(
                big_v.at[nbg], s_sh.at[ebuf_v.at[gp3, 1]], sem_s).wait()
            nxt = jnp.minimum(g + 1, CPW - 1)
            pltpu.sync_copy(pk_hbm.at[cbase + nxt], ebuf_v.at[gn3])
            pltpu.async_copy(
                hp_hbm.at[ebuf_v.at[gn3, 0]], big_v.at[nbg], sem_g)
            _scale(big_v, wring_v, ebuf_v, bg, g3)
            pltpu.async_copy(big_v.at[bg], s_sh.at[ebuf_v.at[g3, 1]], sem_s,
                             add=True)
            return carry

        lax.fori_loop(1, CPW, iter_g, 0)

        bl = (CPW - 1) % 2
        pltpu.make_async_copy(
            big_v.at[bl], s_sh.at[ebuf_v.at[(CPW - 1) % 3, 1]], sem_s).wait()
        pltpu.make_async_copy(
            hp_hbm.at[ebuf_v.at[CPW % 3, 0]], big_v.at[1 - bl], sem_g).wait()
        plsc.subcore_barrier()
        pltpu.sync_copy(s_sh.at[pl.ds(r0, ROWS_PT)],
                        s_out.at[pl.ds(c * N_PAD + r0, ROWS_PT)])

    return pl.kernel(
        body,
        out_type=jax.ShapeDtypeStruct((2 * N_PAD, CW), jnp.float32),
        mesh=_SC_MESH,
        compiler_params=_SC_PARAMS,
        scratch_types=[
            pltpu.VMEM((3, 3, CH), jnp.int32),
            pltpu.VMEM((2, CH, CW), jnp.float32),
            pltpu.VMEM((CH,), jnp.float32),
            pltpu.VMEM_SHARED((N_PAD, CW), jnp.float32),
            pltpu.SemaphoreType.DMA,
            pltpu.SemaphoreType.DMA,
        ],
    )


_sc_agg_1 = _make_sc_agg(D_HID)
_sc_agg_2 = _make_sc_agg(D_OUT)

_dense_a = pl.pallas_call(
    _tc_dense_a,
    out_shape=(
        jax.ShapeDtypeStruct((N_PAD, D_HID + 16), jnp.float32),
        jax.ShapeDtypeStruct((N_PAD,), jnp.float32),
        jax.ShapeDtypeStruct((N_PAD,), jnp.float32),
    ),
)

_mid = pl.pallas_call(
    _tc_mid,
    out_shape=(
        jax.ShapeDtypeStruct((N_PAD, D_OUT + 16), jnp.float32),
        jax.ShapeDtypeStruct((N_PAD,), jnp.float32),
        jax.ShapeDtypeStruct((N_PAD,), jnp.float32),
    ),
)

_final = pl.pallas_call(
    _tc_final,
    out_shape=jax.ShapeDtypeStruct((N_PAD, D_OUT), jnp.float32),
)


def kernel(x, edge_index, W1, att_src1, att_dst1, b1,
           W2, att_src2, att_dst2, b2):
    x = x.astype(jnp.float32)
    loops = jnp.arange(N_NODES, dtype=jnp.int32)
    padv = jnp.full((E_PAD - E_TOT,), N_NODES, dtype=jnp.int32)
    src = jnp.concatenate([edge_index[0].astype(jnp.int32), loops, padv])
    dst = jnp.concatenate([edge_index[1].astype(jnp.int32), loops, padv])

    x_pad = jnp.zeros((N_PAD, D_IN), jnp.float32).at[:N_NODES].set(x)

    h1, as1, ad1 = _dense_a(x_pad, W1, att_src1.reshape(1, D_HID),
                            att_dst1.reshape(1, D_HID))
    pk1 = _sc_w(src, dst, as1, ad1)
    s1 = _sc_agg_1(h1, pk1).reshape(2, N_PAD, D_HID + 16)
    h2, as2, ad2 = _mid(s1, b1.reshape(1, D_HID), W2,
                        att_src2.reshape(1, D_OUT), att_dst2.reshape(1, D_OUT))
    pk2 = _sc_w(src, dst, as2, ad2)
    s2 = _sc_agg_2(h2, pk2).reshape(2, N_PAD, D_OUT + 16)
    out = _final(s2, b2.reshape(1, D_OUT))
    return out[:N_NODES]


# speedup vs baseline: 27.4504x; 1.0805x over previous
import jax
import jax.numpy as jnp
from jax import lax
from jax.experimental import pallas as pl
from jax.experimental.pallas import tpu as pltpu
from jax.experimental.pallas import tpu_sc as plsc

N_NODES = 10000
D_IN = 128
D_HID = 16
D_OUT = 128
N_EDGES = 320000

N_PAD = 10240
E_TOT = N_EDGES + N_NODES
CH = 128
N_WORKERS = 32
CPW = -(-E_TOT // (N_WORKERS * CH))
E_PAD = N_WORKERS * CPW * CH
NCHUNKS = N_WORKERS * CPW
SUP = 9
ROWS_PT = N_PAD // 16

_SC_MESH = plsc.VectorSubcoreMesh(core_axis_name="c", subcore_axis_name="s")
_SC_PARAMS = pltpu.CompilerParams(
    needs_layout_passes=False, use_tc_tiling_on_sc=False)


def _tc_dense_a(x_ref, w_ref, asrc_ref, adst_ref, h_ref, as_ref, ad_ref):
    h = jnp.dot(x_ref[...], w_ref[...], preferred_element_type=jnp.float32)
    h_ref[...] = h
    as_ref[...] = jnp.sum(h * asrc_ref[...], axis=1)
    ad_ref[...] = jnp.sum(h * adst_ref[...], axis=1)


def _tc_mid(s_ref, dn_ref, b1_ref, w2_ref, asrc_ref, adst_ref,
            h2_ref, as_ref, ad_ref):
    s = s_ref[0] + s_ref[1]
    den = jnp.sum(dn_ref[...], axis=0)[:, None] + 1e-16
    hin = jnp.maximum(s / den + b1_ref[...], 0.0)
    h2 = jnp.dot(hin, w2_ref[...], preferred_element_type=jnp.float32)
    h2_ref[...] = h2
    as_ref[...] = jnp.sum(h2 * asrc_ref[...], axis=1)
    ad_ref[...] = jnp.sum(h2 * adst_ref[...], axis=1)


def _tc_final(s_ref, dn_ref, b2_ref, out_ref):
    s = s_ref[0] + s_ref[1]
    den = jnp.sum(dn_ref[...], axis=0)[:, None] + 1e-16
    out_ref[...] = s / den + b2_ref[...]


def _sc_w_body(src_hbm, dst_hbm, as_hbm, ad_hbm, pk_out, dn_out,
               as_v, ad_v, dn_v, sd_v, stage_v):
    c = lax.axis_index("c")
    s = lax.axis_index("s")
    wid = c * 16 + s
    pltpu.sync_copy(as_hbm, as_v)
    pltpu.sync_copy(ad_hbm, ad_v)

    zv = jnp.zeros((16,), jnp.float32)

    def zdn(i, carry):
        dn_v[pl.ds(i * 16, 16)] = zv
        return carry

    lax.fori_loop(0, N_PAD // 16, zdn, 0)

    def sup(s9, carry):
        ebase = wid * (CPW * CH) + s9 * (SUP * CH)
        pltpu.sync_copy(src_hbm.at[pl.ds(ebase, SUP * CH)], sd_v.at[0])
        pltpu.sync_copy(dst_hbm.at[pl.ds(ebase, SUP * CH)], sd_v.at[1])

        def chunk(k, carry2):
            for j in range(CH // 16):
                sv = sd_v[0, pl.ds(k * CH + j * 16, 16)]
                dv = sd_v[1, pl.ds(k * CH + j * 16, 16)]
                v = (plsc.load_gather(as_v, [sv])
                     + plsc.load_gather(ad_v, [dv]))
                v = jnp.where(v >= 0, v, 0.2 * v)
                w = jnp.exp(v)
                plsc.addupdate_scatter(dn_v, [dv], w)
                stage_v[k, 0, pl.ds(j * 16, 16)] = sv
                stage_v[k, 1, pl.ds(j * 16, 16)] = dv
                stage_v[k, 2, pl.ds(j * 16, 16)] = plsc.bitcast(w, jnp.int32)
            return carry2

        lax.fori_loop(0, SUP, chunk, 0)
        pltpu.sync_copy(stage_v, pk_out.at[pl.ds(wid * CPW + s9 * SUP, SUP)])
        return carry

    lax.fori_loop(0, CPW // SUP, sup, 0)
    pltpu.sync_copy(dn_v, dn_out.at[wid])


_sc_w = pl.kernel(
    _sc_w_body,
    out_type=(
        jax.ShapeDtypeStruct((NCHUNKS, 3, CH), jnp.int32),
        jax.ShapeDtypeStruct((N_WORKERS, N_PAD), jnp.float32),
    ),
    mesh=_SC_MESH,
    compiler_params=_SC_PARAMS,
    scratch_types=[
        pltpu.VMEM((N_PAD,), jnp.float32),
        pltpu.VMEM((N_PAD,), jnp.float32),
        pltpu.VMEM((N_PAD,), jnp.float32),
        pltpu.VMEM((2, SUP * CH), jnp.int32),
        pltpu.VMEM((SUP, 3, CH), jnp.int32),
    ],
)


def _make_sc_agg(C):
    def _scale(big_v, wring_v, ebuf_v, bg, g3):
        for j in range(CH // 16):
            wring_v[pl.ds(j * 16, 16)] = plsc.bitcast(
                ebuf_v[g3, 2, pl.ds(j * 16, 16)], jnp.float32)

        def grp(j2, carry):
            for l in range(16):
                e = j2 * 16 + l
                ws = plsc.load_gather(
                    wring_v, [jnp.full((16,), e, jnp.int32)])
                for k in range(C // 16):
                    big_v[bg, e, pl.ds(k * 16, 16)] = (
                        big_v[bg, e, pl.ds(k * 16, 16)] * ws)
            return carry

        lax.fori_loop(0, CH // 16, grp, 0)

    def body(hp_hbm, pk_hbm, s_out,
             ebuf_v, big_v, wring_v, s_sh, sem_g, sem_s):
        c = lax.axis_index("c")
        s = lax.axis_index("s")
        wid = c * 16 + s
        cbase = wid * CPW
        r0 = s * ROWS_PT

        pltpu.sync_copy(pk_hbm.at[cbase], ebuf_v.at[0])
        pltpu.async_copy(hp_hbm.at[ebuf_v.at[0, 0]], big_v.at[0], sem_g)

        zv = jnp.zeros((16,), jnp.float32)

        def zrow(i, carry):
            for j in range(C // 16):
                big_v[1, i, pl.ds(j * 16, 16)] = zv
            return carry

        lax.fori_loop(0, CH, zrow, 0)
        for i in range(ROWS_PT // CH):
            pltpu.sync_copy(big_v.at[1], s_sh.at[pl.ds(r0 + i * CH, CH)])
        plsc.subcore_barrier()

        pltpu.make_async_copy(
            hp_hbm.at[ebuf_v.at[0, 0]], big_v.at[0], sem_g).wait()
        pltpu.sync_copy(pk_hbm.at[cbase + 1], ebuf_v.at[1])
        pltpu.async_copy(hp_hbm.at[ebuf_v.at[1, 0]], big_v.at[1], sem_g)
        _scale(big_v, wring_v, ebuf_v, 0, 0)
        pltpu.async_copy(big_v.at[0], s_sh.at[ebuf_v.at[0, 1]], sem_s,
                         add=True)

        def iter_g(g, carry):
            bg = lax.rem(g, 2)
            nbg = 1 - bg
            g3 = lax.rem(g, 3)
            gn3 = lax.rem(g + 1, 3)
            gp3 = lax.rem(g + 2, 3)
            pltpu.make_async_copy(
                hp_hbm.at[ebuf_v.at[g3, 0]], big_v.at[bg], sem_g).wait()
            pltpu.make_async_copy(
                big_v.at[nbg], s_sh.at[ebuf_v.at[gp3, 1]], sem_s).wait()
            nxt = jnp.minimum(g + 1, CPW - 1)
            pltpu.sync_copy(pk_hbm.at[cbase + nxt], ebuf_v.at[gn3])
            pltpu.async_copy(
                hp_hbm.at[ebuf_v.at[gn3, 0]], big_v.at[nbg], sem_g)
            _scale(big_v, wring_v, ebuf_v, bg, g3)
            pltpu.async_copy(big_v.at[bg], s_sh.at[ebuf_v.at[g3, 1]], sem_s,
                             add=True)
            return carry

        lax.fori_loop(1, CPW, iter_g, 0)

        bl = (CPW - 1) % 2
        pltpu.make_async_copy(
            big_v.at[bl], s_sh.at[ebuf_v.at[(CPW - 1) % 3, 1]], sem_s).wait()
        pltpu.make_async_copy(
            hp_hbm.at[ebuf_v.at[CPW % 3, 0]], big_v.at[1 - bl], sem_g).wait()
        plsc.subcore_barrier()
        pltpu.sync_copy(s_sh.at[pl.ds(r0, ROWS_PT)],
                        s_out.at[pl.ds(c * N_PAD + r0, ROWS_PT)])

    return pl.kernel(
        body,
        out_type=jax.ShapeDtypeStruct((2 * N_PAD, C), jnp.float32),
        mesh=_SC_MESH,
        compiler_params=_SC_PARAMS,
        scratch_types=[
            pltpu.VMEM((3, 3, CH), jnp.int32),
            pltpu.VMEM((2, CH, C), jnp.float32),
            pltpu.VMEM((CH,), jnp.float32),
            pltpu.VMEM_SHARED((N_PAD, C), jnp.float32),
            pltpu.SemaphoreType.DMA,
            pltpu.SemaphoreType.DMA,
        ],
    )


_sc_agg_1 = _make_sc_agg(D_HID)
_sc_agg_2 = _make_sc_agg(D_OUT)

_dense_a = pl.pallas_call(
    _tc_dense_a,
    out_shape=(
        jax.ShapeDtypeStruct((N_PAD, D_HID), jnp.float32),
        jax.ShapeDtypeStruct((N_PAD,), jnp.float32),
        jax.ShapeDtypeStruct((N_PAD,), jnp.float32),
    ),
)

_mid = pl.pallas_call(
    _tc_mid,
    out_shape=(
        jax.ShapeDtypeStruct((N_PAD, D_OUT), jnp.float32),
        jax.ShapeDtypeStruct((N_PAD,), jnp.float32),
        jax.ShapeDtypeStruct((N_PAD,), jnp.float32),
    ),
)

_final = pl.pallas_call(
    _tc_final,
    out_shape=jax.ShapeDtypeStruct((N_PAD, D_OUT), jnp.float32),
)


def kernel(x, edge_index, W1, att_src1, att_dst1, b1,
           W2, att_src2, att_dst2, b2):
    x = x.astype(jnp.float32)
    loops = jnp.arange(N_NODES, dtype=jnp.int32)
    padv = jnp.full((E_PAD - E_TOT,), N_NODES, dtype=jnp.int32)
    src = jnp.concatenate([edge_index[0].astype(jnp.int32), loops, padv])
    dst = jnp.concatenate([edge_index[1].astype(jnp.int32), loops, padv])

    x_pad = jnp.zeros((N_PAD, D_IN), jnp.float32).at[:N_NODES].set(x)

    h1, as1, ad1 = _dense_a(x_pad, W1, att_src1.reshape(1, D_HID),
                            att_dst1.reshape(1, D_HID))
    pk1, dn1 = _sc_w(src, dst, as1, ad1)
    s1 = _sc_agg_1(h1, pk1).reshape(2, N_PAD, D_HID)
    h2, as2, ad2 = _mid(s1, dn1, b1.reshape(1, D_HID), W2,
                        att_src2.reshape(1, D_OUT), att_dst2.reshape(1, D_OUT))
    pk2, dn2 = _sc_w(src, dst, as2, ad2)
    s2 = _sc_agg_2(h2, pk2).reshape(2, N_PAD, D_OUT)
    out = _final(s2, dn2, b2.reshape(1, D_OUT))
    return out[:N_NODES]
